# Initial kernel scaffold; baseline (speedup 1.0000x reference)
import jax
import jax.numpy as jnp
from jax.experimental import pallas as pl


def kernel(x, Wr, W1, b1, W2, b2):
    raise NotImplementedError("write your pallas kernel here")


# speedup vs baseline: 1.1713x; 1.1713x over previous
import functools

import jax
import jax.numpy as jnp
from jax import lax
from jax.experimental import pallas as pl
from jax.experimental.pallas import tpu as pltpu
from jax.experimental.pallas import tpu_sc as plsc

E = 8
K = 2
CAP_FACTOR = 1.25
LANES = 128


def _dyn_gather(vec, idx):
    return lax.gather(
        vec, idx[:, None],
        lax.GatherDimensionNumbers(
            offset_dims=(), collapsed_slice_dims=(0,), start_index_map=(0,)),
        slice_sizes=(1,),
        mode=lax.GatherScatterMode.PROMISE_IN_BOUNDS)


def _router_body(x_ref, w_ref, o_ref, *, e):
    logits = jnp.dot(x_ref[...], w_ref[...], preferred_element_type=jnp.float32)
    lane = lax.broadcasted_iota(jnp.int32, logits.shape, 1)
    valid = lane < e
    z = jnp.where(valid, logits, -1e30)
    m = jnp.max(z, axis=1, keepdims=True)
    p = jnp.exp(z - m)
    p = jnp.where(valid, p, 0.0)
    s = jnp.sum(p, axis=1, keepdims=True)
    o_ref[...] = p / s


def _router(xf, wr_pad, blk=512):
    n, c = xf.shape
    return pl.pallas_call(
        functools.partial(_router_body, e=E),
        grid=(n // blk,),
        in_specs=[
            pl.BlockSpec((blk, c), lambda i: (i, 0)),
            pl.BlockSpec((c, LANES), lambda i: (0, 0)),
        ],
        out_specs=pl.BlockSpec((blk, LANES), lambda i: (i, 0)),
        out_shape=jax.ShapeDtypeStruct((n, LANES), jnp.float32),
    )(xf, wr_pad)


def _routing_dispatch(probs, xf, *, n_tok, n_rows, cap, t_len, buf_rows):
    c_dim = xf.shape[1]
    n_sub = 16
    tpt = n_tok // n_sub
    apt = tpt * K
    nv = apt // 16
    n_groups = E * (n_tok // t_len)
    b_rows = n_tok // t_len

    dump_n = buf_rows - n_rows
    rpt = buf_rows // n_sub
    nrv = rpt // 16

    mesh = plsc.VectorSubcoreMesh(
        core_axis_name="c", subcore_axis_name="s", num_cores=1)

    @functools.partial(
        pl.kernel,
        out_type=(
            jax.ShapeDtypeStruct((buf_rows, c_dim), jnp.float32),
            jax.ShapeDtypeStruct((n_tok * K,), jnp.int32),
            jax.ShapeDtypeStruct((n_tok * K,), jnp.float32),
        ),
        mesh=mesh,
        scratch_types=(
            pltpu.VMEM((tpt * LANES,), jnp.float32),
            pltpu.VMEM((apt,), jnp.int32),
            pltpu.VMEM((apt,), jnp.float32),
            pltpu.VMEM((apt,), jnp.int32),
            pltpu.VMEM((apt,), jnp.int32),
            pltpu.VMEM((16,), jnp.int32),
            pltpu.VMEM_SHARED((n_sub * 16,), jnp.int32),
            pltpu.VMEM((n_sub * 16,), jnp.int32),
            pltpu.VMEM((buf_rows,), jnp.int32),
            pltpu.VMEM_SHARED((n_sub * buf_rows,), jnp.int32),
            pltpu.VMEM((rpt,), jnp.int32),
            pltpu.VMEM((rpt,), jnp.int32),
            pltpu.VMEM((16, c_dim), jnp.float32),
            pltpu.VMEM((apt,), jnp.int32),
            pltpu.VMEM((apt,), jnp.float32),
            pltpu.SemaphoreType.DMA,
        ),
        compiler_params=pltpu.CompilerParams(needs_layout_passes=False),
    )
    def k(probs_hbm, x_hbm, buf_hbm, cidx_hbm, cw_hbm,
          probs_v, ea_v, pa_v, g_v, lp_v, cnt_v, hist_sh, hist_v,
          ids_v, table_sh, seg_v, mseg_v, row_v, ci_v, cwt_v, sem_g):
        sid = lax.axis_index("s")
        t0 = sid * tpt
        iota = lax.iota(jnp.int32, 16)

        pltpu.sync_copy(probs_hbm.at[pl.ds(t0 * LANES, tpt * LANES)], probs_v)

        def top2_body(j, carry):
            r_idx = iota + j * 16
            ps = [plsc.load_gather(probs_v, [r_idx * LANES + e])
                  for e in range(E)]
            m1 = ps[0]
            for e in range(1, E):
                m1 = jnp.maximum(m1, ps[e])
            i1 = jnp.full((16,), E, jnp.int32)
            for e in range(E):
                i1 = jnp.minimum(i1, jnp.where(ps[e] == m1, e, E))
            m2 = jnp.full((16,), -1.0, jnp.float32)
            for e in range(E):
                pe = jnp.where(i1 == e, -1.0, ps[e])
                m2 = jnp.maximum(m2, pe)
            i2 = jnp.full((16,), E, jnp.int32)
            for e in range(E):
                pe = jnp.where(i1 == e, -1.0, ps[e])
                i2 = jnp.minimum(i2, jnp.where(pe == m2, e, E))
            a_even = 2 * r_idx
            plsc.store_scatter(ea_v, [a_even], i1)
            plsc.store_scatter(ea_v, [a_even + 1], i2)
            plsc.store_scatter(pa_v, [a_even], m1)
            plsc.store_scatter(pa_v, [a_even + 1], m2)
            return carry

        lax.fori_loop(0, tpt // 16, top2_body, 0)

        def rank_body(v, cnt):
            ea = ea_v[pl.ds(v * 16, 16)]
            aid = v * 16 + iota
            tok = t0 + aid // K
            row = tok // t_len
            g = ea * b_rows + row
            lpos = jnp.zeros((16,), jnp.int32)
            for gv in range(n_groups):
                m = g == gv
                ones = jnp.where(m, 1, 0)
                cs = plsc.cumsum(ones)
                cnt_gv = jnp.sum(jnp.where(iota == gv, cnt, 0))
                lpos = jnp.where(m, cnt_gv + cs - 1, lpos)
                cnt = cnt + jnp.where(iota == gv, jnp.sum(ones), 0)
            g_v[pl.ds(v * 16, 16)] = g
            lp_v[pl.ds(v * 16, 16)] = lpos
            return cnt

        cnt = lax.fori_loop(0, nv, rank_body, jnp.zeros((16,), jnp.int32))
        cnt_v[...] = cnt

        pltpu.sync_copy(cnt_v, hist_sh.at[pl.ds(sid * 16, 16)])
        plsc.subcore_barrier()
        pltpu.sync_copy(hist_sh, hist_v)

        def base_body(sp, base):
            h = hist_v[pl.ds(sp * 16, 16)]
            return jnp.where(sp < sid, base + h, base)

        base = lax.fori_loop(0, n_sub, base_body, jnp.zeros((16,), jnp.int32))

        neg1 = jnp.full((16,), -1, jnp.int32)

        def clear_body(i, carry):
            ids_v[pl.ds(i * 16, 16)] = neg1
            return carry

        lax.fori_loop(0, buf_rows // 16, clear_body, 0)

        def meta_body(v, carry):
            aid = v * 16 + iota
            tok = t0 + aid // K
            g = g_v[pl.ds(v * 16, 16)]
            lpos = lp_v[pl.ds(v * 16, 16)]
            pa = pa_v[pl.ds(v * 16, 16)]
            gbase = _dyn_gather(base, g)
            gpos = gbase + lpos
            keep = gpos < cap
            slot = jnp.where(keep, g * cap + gpos, n_rows + (aid % dump_n))
            ci_v[pl.ds(v * 16, 16)] = jnp.where(keep, slot, 0)
            cwt_v[pl.ds(v * 16, 16)] = jnp.where(keep, pa, 0.0)
            plsc.store_scatter(ids_v, [slot], tok)
            return carry

        lax.fori_loop(0, nv, meta_body, 0)

        pltpu.sync_copy(ci_v, cidx_hbm.at[pl.ds(sid * apt, apt)])
        pltpu.sync_copy(cwt_v, cw_hbm.at[pl.ds(sid * apt, apt)])

        pltpu.sync_copy(ids_v, table_sh.at[pl.ds(sid * buf_rows, buf_rows)])
        plsc.subcore_barrier()

        r0 = sid * rpt

        def clear2_body(i, carry):
            mseg_v[pl.ds(i * 16, 16)] = neg1
            return carry

        lax.fori_loop(0, nrv, clear2_body, 0)

        def merge_body(sp, carry):
            pltpu.sync_copy(table_sh.at[pl.ds(sp * buf_rows + r0, rpt)], seg_v)

            def mx_body(i, carry2):
                o = i * 16
                mseg_v[pl.ds(o, 16)] = jnp.maximum(
                    mseg_v[pl.ds(o, 16)], seg_v[pl.ds(o, 16)])
                return carry2

            lax.fori_loop(0, nrv, mx_body, 0)
            return carry

        lax.fori_loop(0, n_sub, merge_body, 0)

        def disp_body(i, carry):
            o = i * 16
            idv = mseg_v[pl.ds(o, 16)]
            mseg_v[pl.ds(o, 16)] = jnp.minimum(jnp.maximum(idv, 0), n_tok - 1)
            pltpu.async_copy(x_hbm.at[mseg_v.at[pl.ds(o, 16)]],
                             row_v, sem_g).wait()
            pltpu.sync_copy(row_v, buf_hbm.at[pl.ds(r0 + o, 16)])
            return carry

        lax.fori_loop(0, nrv, disp_body, 0)

    return k(probs, xf)


def _ffn_body(buf_ref, w1_ref, b1_ref, w2_ref, b2_ref, o_ref, acc_ref, *, hb_n):
    hb = pl.program_id(1)

    @pl.when(hb == 0)
    def _():
        acc_ref[...] = jnp.broadcast_to(b2_ref[0], acc_ref.shape)

    h = jnp.dot(buf_ref[...], w1_ref[0], preferred_element_type=jnp.float32)
    h = jax.nn.relu(h + b1_ref[0])
    acc_ref[...] += jnp.dot(h, w2_ref[0], preferred_element_type=jnp.float32)

    @pl.when(hb == hb_n - 1)
    def _():
        o_ref[...] = acc_ref[...]


def _ffn(buf, w1, b1, w2, b2, *, n_rows, h_blk=512):
    c = buf.shape[1]
    h_dim = w1.shape[2]
    hb_n = h_dim // h_blk
    rpe = n_rows // E
    return pl.pallas_call(
        functools.partial(_ffn_body, hb_n=hb_n),
        grid=(E, hb_n),
        in_specs=[
            pl.BlockSpec((rpe, c), lambda e, hb: (e, 0)),
            pl.BlockSpec((1, c, h_blk), lambda e, hb: (e, 0, hb)),
            pl.BlockSpec((1, 1, h_blk), lambda e, hb: (e, 0, hb)),
            pl.BlockSpec((1, h_blk, c), lambda e, hb: (e, hb, 0)),
            pl.BlockSpec((1, 1, c), lambda e, hb: (e, 0, 0)),
        ],
        out_specs=pl.BlockSpec((rpe, c), lambda e, hb: (e, 0)),
        out_shape=jax.ShapeDtypeStruct((n_rows, c), jnp.float32),
        scratch_shapes=[pltpu.VMEM((rpe, c), jnp.float32)],
        compiler_params=pltpu.CompilerParams(
            dimension_semantics=("arbitrary", "arbitrary")),
    )(buf, w1, b1, w2, b2)


def _combine(yb, cidx, cw, *, n_tok):
    c_dim = yb.shape[1]
    n_workers = 32
    tpw = n_tok // n_workers
    apw = tpw * K
    cv = c_dim // 16

    mesh = plsc.VectorSubcoreMesh(core_axis_name="c", subcore_axis_name="s")

    @functools.partial(
        pl.kernel,
        out_type=jax.ShapeDtypeStruct((n_tok, c_dim), jnp.float32),
        mesh=mesh,
        scratch_types=(
            pltpu.VMEM((apw,), jnp.int32),
            pltpu.VMEM((apw,), jnp.float32),
            pltpu.VMEM((32, c_dim), jnp.float32),
            pltpu.VMEM((16, c_dim), jnp.float32),
            pltpu.SemaphoreType.DMA,
        ),
        compiler_params=pltpu.CompilerParams(needs_layout_passes=False),
    )
    def k(yb_hbm, ci_hbm, cw_hbm, out_hbm, ci_v, cw_v, rows_v, out_v, sem):
        cid = lax.axis_index("c")
        sid = lax.axis_index("s")
        wid = cid * 16 + sid
        a0 = wid * apw
        t0 = wid * tpw
        iota = lax.iota(jnp.int32, 16)

        pltpu.sync_copy(ci_hbm.at[pl.ds(a0, apw)], ci_v)
        pltpu.sync_copy(cw_hbm.at[pl.ds(a0, apw)], cw_v)

        def chunk_body(j, carry):
            pltpu.async_copy(
                yb_hbm.at[ci_v.at[pl.ds(j * 32, 32)]], rows_v, sem).wait()
            we = plsc.load_gather(cw_v, [j * 32 + 2 * iota])
            wo = plsc.load_gather(cw_v, [j * 32 + 2 * iota + 1])

            def tok_body(tt, carry2):
                ttv = jnp.full((16,), tt, jnp.int32)
                w0 = _dyn_gather(we, ttv)
                w1 = _dyn_gather(wo, ttv)
                m0 = w0 != 0.0
                m1 = w1 != 0.0
                for cc in range(cv):
                    r0 = rows_v[2 * tt, pl.ds(cc * 16, 16)]
                    r1 = rows_v[2 * tt + 1, pl.ds(cc * 16, 16)]
                    o = jnp.where(m0, w0 * r0, 0.0) + jnp.where(m1, w1 * r1, 0.0)
                    out_v[tt, pl.ds(cc * 16, 16)] = o
                return carry2

            lax.fori_loop(0, 16, tok_body, 0)
            pltpu.sync_copy(out_v, out_hbm.at[pl.ds(t0 + j * 16, 16)])
            return carry

        lax.fori_loop(0, tpw // 16, chunk_body, 0)

    return k(yb, cidx, cw)


def kernel(x, Wr, W1, b1, W2, b2):
    b, t, c = x.shape
    n = b * t
    cap = int(t / E * CAP_FACTOR)
    n_rows = E * b * cap
    buf_rows = n_rows + 256

    xf = x.reshape(n, c)
    wr_pad = jnp.pad(Wr, ((0, 0), (0, LANES - E)))

    probs = _router(xf, wr_pad)
    buf, cidx, cw = _routing_dispatch(
        probs.reshape(-1), xf, n_tok=n, n_rows=n_rows, cap=cap, t_len=t,
        buf_rows=buf_rows)
    yb = _ffn(buf[:n_rows], W1, b1.reshape(E, 1, -1), W2, b2.reshape(E, 1, -1),
              n_rows=n_rows)
    out = _combine(yb, cidx, cw, n_tok=n)
    return out.reshape(b, t, c)


# speedup vs baseline: 1.5947x; 1.3615x over previous
import functools

import jax
import jax.numpy as jnp
from jax import lax
from jax.experimental import pallas as pl
from jax.experimental.pallas import tpu as pltpu
from jax.experimental.pallas import tpu_sc as plsc

E = 8
K = 2
CAP_FACTOR = 1.25
LANES = 128


def _dyn_gather(vec, idx):
    return lax.gather(
        vec, idx[:, None],
        lax.GatherDimensionNumbers(
            offset_dims=(), collapsed_slice_dims=(0,), start_index_map=(0,)),
        slice_sizes=(1,),
        mode=lax.GatherScatterMode.PROMISE_IN_BOUNDS)


def _router_body(x_ref, w_ref, o_ref, *, e):
    logits = jnp.dot(x_ref[...], w_ref[...], preferred_element_type=jnp.float32)
    lane = lax.broadcasted_iota(jnp.int32, logits.shape, 1)
    valid = lane < e
    z = jnp.where(valid, logits, -1e30)
    m = jnp.max(z, axis=1, keepdims=True)
    p = jnp.exp(z - m)
    p = jnp.where(valid, p, 0.0)
    s = jnp.sum(p, axis=1, keepdims=True)
    o_ref[...] = p / s


def _router(xf, wr_pad, blk=512):
    n, c = xf.shape
    return pl.pallas_call(
        functools.partial(_router_body, e=E),
        grid=(n // blk,),
        in_specs=[
            pl.BlockSpec((blk, c), lambda i: (i, 0)),
            pl.BlockSpec((c, LANES), lambda i: (0, 0)),
        ],
        out_specs=pl.BlockSpec((blk, LANES), lambda i: (i, 0)),
        out_shape=jax.ShapeDtypeStruct((n, LANES), jnp.float32),
    )(xf, wr_pad)


def _routing_dispatch(probs, xf, *, n_tok, n_rows, cap, t_len, buf_rows):
    c_dim = xf.shape[1]
    n_sub = 16
    tpt = n_tok // n_sub
    apt = tpt * K
    nv = apt // 16
    n_groups = E * (n_tok // t_len)
    b_rows = n_tok // t_len
    dump_n = E * b_rows * cap // 8
    rpt = buf_rows // n_sub
    nrv = rpt // 16

    mesh = plsc.VectorSubcoreMesh(
        core_axis_name="c", subcore_axis_name="s", num_cores=1)

    @functools.partial(
        pl.kernel,
        out_type=(
            jax.ShapeDtypeStruct((buf_rows, c_dim), jnp.float32),
            jax.ShapeDtypeStruct((n_tok,), jnp.int32),
            jax.ShapeDtypeStruct((n_tok,), jnp.int32),
            jax.ShapeDtypeStruct((buf_rows,), jnp.float32),
        ),
        mesh=mesh,
        scratch_types=(
            pltpu.VMEM((tpt * LANES,), jnp.float32),
            pltpu.VMEM((apt,), jnp.int32),
            pltpu.VMEM((apt,), jnp.float32),
            pltpu.VMEM((apt,), jnp.int32),
            pltpu.VMEM((apt,), jnp.int32),
            pltpu.VMEM((apt,), jnp.int32),
            pltpu.VMEM((16,), jnp.int32),
            pltpu.VMEM_SHARED((n_sub * 16,), jnp.int32),
            pltpu.VMEM((n_sub * 16,), jnp.int32),
            pltpu.VMEM((buf_rows,), jnp.int32),
            pltpu.VMEM((buf_rows,), jnp.float32),
            pltpu.VMEM_SHARED((n_sub * buf_rows,), jnp.int32),
            pltpu.VMEM_SHARED((n_sub * buf_rows,), jnp.float32),
            pltpu.VMEM((rpt,), jnp.int32),
            pltpu.VMEM((rpt,), jnp.float32),
            pltpu.VMEM((rpt,), jnp.int32),
            pltpu.VMEM((rpt,), jnp.float32),
            pltpu.VMEM((2, 16, c_dim), jnp.float32),
            pltpu.VMEM((tpt,), jnp.int32),
            pltpu.VMEM((tpt,), jnp.int32),
            pltpu.SemaphoreType.DMA,
            pltpu.SemaphoreType.DMA,
        ),
        compiler_params=pltpu.CompilerParams(needs_layout_passes=False),
    )
    def k(probs_hbm, x_hbm, buf_hbm, cie_hbm, cio_hbm, w_hbm,
          probs_v, ea_v, pa_v, g_v, lp_v, slot_st, cnt_v, hist_sh, hist_v,
          ids_v, wtab_v, ttok_sh, tw_sh, seg_v, wseg_v, mseg_v, mwseg_v,
          row_v, cie_v, cio_v, sem_a, sem_b):
        sid = lax.axis_index("s")
        t0 = sid * tpt
        iota = lax.iota(jnp.int32, 16)

        pltpu.sync_copy(probs_hbm.at[pl.ds(t0 * LANES, tpt * LANES)], probs_v)

        def top2_body(j, carry):
            r_idx = iota + j * 16
            ps = [plsc.load_gather(probs_v, [r_idx * LANES + e])
                  for e in range(E)]
            m1 = ps[0]
            for e in range(1, E):
                m1 = jnp.maximum(m1, ps[e])
            i1 = jnp.full((16,), E, jnp.int32)
            for e in range(E):
                i1 = jnp.minimum(i1, jnp.where(ps[e] == m1, e, E))
            m2 = jnp.full((16,), -1.0, jnp.float32)
            for e in range(E):
                pe = jnp.where(i1 == e, -1.0, ps[e])
                m2 = jnp.maximum(m2, pe)
            i2 = jnp.full((16,), E, jnp.int32)
            for e in range(E):
                pe = jnp.where(i1 == e, -1.0, ps[e])
                i2 = jnp.minimum(i2, jnp.where(pe == m2, e, E))
            a_even = 2 * r_idx
            plsc.store_scatter(ea_v, [a_even], i1)
            plsc.store_scatter(ea_v, [a_even + 1], i2)
            plsc.store_scatter(pa_v, [a_even], m1)
            plsc.store_scatter(pa_v, [a_even + 1], m2)
            return carry

        lax.fori_loop(0, tpt // 16, top2_body, 0)

        def rank_body(v, cnt):
            ea = ea_v[pl.ds(v * 16, 16)]
            aid = v * 16 + iota
            tok = t0 + aid // K
            row = tok // t_len
            g = ea * b_rows + row
            lpos = jnp.zeros((16,), jnp.int32)
            for gv in range(n_groups):
                m = g == gv
                ones = jnp.where(m, 1, 0)
                cs = plsc.cumsum(ones)
                cnt_gv = jnp.sum(jnp.where(iota == gv, cnt, 0))
                lpos = jnp.where(m, cnt_gv + cs - 1, lpos)
                cnt = cnt + jnp.where(iota == gv, jnp.sum(ones), 0)
            g_v[pl.ds(v * 16, 16)] = g
            lp_v[pl.ds(v * 16, 16)] = lpos
            return cnt

        cnt = lax.fori_loop(0, nv, rank_body, jnp.zeros((16,), jnp.int32))
        cnt_v[...] = cnt

        pltpu.sync_copy(cnt_v, hist_sh.at[pl.ds(sid * 16, 16)])
        plsc.subcore_barrier()
        pltpu.sync_copy(hist_sh, hist_v)

        def base_body(sp, base):
            h = hist_v[pl.ds(sp * 16, 16)]
            return jnp.where(sp < sid, base + h, base)

        base = lax.fori_loop(0, n_sub, base_body, jnp.zeros((16,), jnp.int32))

        neg1 = jnp.full((16,), -1, jnp.int32)
        neg1f = jnp.full((16,), -1.0, jnp.float32)

        def clear_body(i, carry):
            ids_v[pl.ds(i * 16, 16)] = neg1
            wtab_v[pl.ds(i * 16, 16)] = neg1f
            return carry

        lax.fori_loop(0, buf_rows // 16, clear_body, 0)

        def meta_body(v, carry):
            aid = v * 16 + iota
            tok = t0 + aid // K
            g = g_v[pl.ds(v * 16, 16)]
            lpos = lp_v[pl.ds(v * 16, 16)]
            pa = pa_v[pl.ds(v * 16, 16)]
            gbase = _dyn_gather(base, g)
            gpos = gbase + lpos
            keep = gpos < cap
            slot = jnp.where(keep, g * cap + gpos, n_rows + (aid % dump_n))
            slot_st[pl.ds(v * 16, 16)] = slot
            plsc.store_scatter(ids_v, [slot], tok)
            plsc.store_scatter(wtab_v, [slot], jnp.where(keep, pa, 0.0))
            return carry

        lax.fori_loop(0, nv, meta_body, 0)

        def sep_body(j, carry):
            tl = j * 16 + iota
            cie_v[pl.ds(j * 16, 16)] = plsc.load_gather(slot_st, [2 * tl])
            cio_v[pl.ds(j * 16, 16)] = plsc.load_gather(slot_st, [2 * tl + 1])
            return carry

        lax.fori_loop(0, tpt // 16, sep_body, 0)
        pltpu.sync_copy(cie_v, cie_hbm.at[pl.ds(sid * tpt, tpt)])
        pltpu.sync_copy(cio_v, cio_hbm.at[pl.ds(sid * tpt, tpt)])

        pltpu.sync_copy(ids_v, ttok_sh.at[pl.ds(sid * buf_rows, buf_rows)])
        pltpu.sync_copy(wtab_v, tw_sh.at[pl.ds(sid * buf_rows, buf_rows)])
        plsc.subcore_barrier()

        r0 = sid * rpt

        def clear2_body(i, carry):
            mseg_v[pl.ds(i * 16, 16)] = neg1
            mwseg_v[pl.ds(i * 16, 16)] = neg1f
            return carry

        lax.fori_loop(0, nrv, clear2_body, 0)

        def merge_body(sp, carry):
            pltpu.sync_copy(ttok_sh.at[pl.ds(sp * buf_rows + r0, rpt)], seg_v)
            pltpu.sync_copy(tw_sh.at[pl.ds(sp * buf_rows + r0, rpt)], wseg_v)

            def mx_body(i, carry2):
                o = i * 16
                mseg_v[pl.ds(o, 16)] = jnp.maximum(
                    mseg_v[pl.ds(o, 16)], seg_v[pl.ds(o, 16)])
                mwseg_v[pl.ds(o, 16)] = jnp.maximum(
                    mwseg_v[pl.ds(o, 16)], wseg_v[pl.ds(o, 16)])
                return carry2

            lax.fori_loop(0, nrv, mx_body, 0)
            return carry

        lax.fori_loop(0, n_sub, merge_body, 0)

        def fin_body(i, carry):
            o = i * 16
            idv = mseg_v[pl.ds(o, 16)]
            mseg_v[pl.ds(o, 16)] = jnp.minimum(jnp.maximum(idv, 0), n_tok - 1)
            mwseg_v[pl.ds(o, 16)] = jnp.maximum(mwseg_v[pl.ds(o, 16)], 0.0)
            return carry

        lax.fori_loop(0, nrv, fin_body, 0)
        pltpu.sync_copy(mwseg_v, w_hbm.at[pl.ds(sid * rpt, rpt)])

        def gather_chunk(i, buf_slot, sem):
            return pltpu.async_copy(
                x_hbm.at[mseg_v.at[pl.ds(i * 16, 16)]], row_v.at[buf_slot], sem)

        def disp_body(io, carry):
            i0 = io * 2
            i1 = io * 2 + 1
            d0 = gather_chunk(i0, 0, sem_a)
            d1 = gather_chunk(i1, 1, sem_b)
            d0.wait()
            pltpu.sync_copy(row_v.at[0], buf_hbm.at[pl.ds(r0 + i0 * 16, 16)])
            d1.wait()
            pltpu.sync_copy(row_v.at[1], buf_hbm.at[pl.ds(r0 + i1 * 16, 16)])
            return carry

        lax.fori_loop(0, nrv // 2, disp_body, 0)
        dt = gather_chunk(nrv - 1, 0, sem_a)
        dt.wait()
        pltpu.sync_copy(row_v.at[0],
                        buf_hbm.at[pl.ds(r0 + (nrv - 1) * 16, 16)])

    return k(probs, xf)


def _ffn_body(buf_ref, w1_ref, b1_ref, w2_ref, b2_ref, ws_ref, o_ref, acc_ref,
              *, hb_n):
    hb = pl.program_id(1)

    @pl.when(hb == 0)
    def _():
        acc_ref[...] = jnp.broadcast_to(b2_ref[0], acc_ref.shape)

    h = jnp.dot(buf_ref[...], w1_ref[0], preferred_element_type=jnp.float32)
    h = jax.nn.relu(h + b1_ref[0])
    acc_ref[...] += jnp.dot(h, w2_ref[0], preferred_element_type=jnp.float32)

    @pl.when(hb == hb_n - 1)
    def _():
        o_ref[...] = acc_ref[...] * ws_ref[...]


def _ffn(buf, w1, b1, w2, b2, w_slot, *, yb_rows, h_blk=512):
    c = buf.shape[1]
    h_dim = w1.shape[2]
    hb_n = h_dim // h_blk
    rpe = yb_rows // (E + 1)
    return pl.pallas_call(
        functools.partial(_ffn_body, hb_n=hb_n),
        grid=(E + 1, hb_n),
        in_specs=[
            pl.BlockSpec((rpe, c), lambda e, hb: (e, 0)),
            pl.BlockSpec((1, c, h_blk),
                         lambda e, hb: (jnp.minimum(e, E - 1), 0, hb)),
            pl.BlockSpec((1, 1, h_blk),
                         lambda e, hb: (jnp.minimum(e, E - 1), 0, hb)),
            pl.BlockSpec((1, h_blk, c),
                         lambda e, hb: (jnp.minimum(e, E - 1), hb, 0)),
            pl.BlockSpec((1, 1, c),
                         lambda e, hb: (jnp.minimum(e, E - 1), 0, 0)),
            pl.BlockSpec((rpe, 1), lambda e, hb: (e, 0)),
        ],
        out_specs=pl.BlockSpec((rpe, c), lambda e, hb: (e, 0)),
        out_shape=jax.ShapeDtypeStruct((yb_rows, c), jnp.float32),
        scratch_shapes=[pltpu.VMEM((rpe, c), jnp.float32)],
        compiler_params=pltpu.CompilerParams(
            dimension_semantics=("arbitrary", "arbitrary")),
    )(buf, w1, b1, w2, b2, w_slot)


def _combine(yb, cie, cio, *, n_tok):
    c_dim = yb.shape[1]
    n_workers = 32
    tpw = n_tok // n_workers
    nj = tpw // 16

    mesh = plsc.VectorSubcoreMesh(core_axis_name="c", subcore_axis_name="s")

    @functools.partial(
        pl.kernel,
        out_type=jax.ShapeDtypeStruct((n_tok, c_dim), jnp.float32),
        mesh=mesh,
        scratch_types=(
            pltpu.VMEM((tpw,), jnp.int32),
            pltpu.VMEM((tpw,), jnp.int32),
            pltpu.VMEM((2, 16, c_dim), jnp.float32),
            pltpu.VMEM((2, 16, c_dim), jnp.float32),
            pltpu.SemaphoreType.DMA,
            pltpu.SemaphoreType.DMA,
            pltpu.SemaphoreType.DMA,
            pltpu.SemaphoreType.DMA,
        ),
        compiler_params=pltpu.CompilerParams(needs_layout_passes=False),
    )
    def k(yb_hbm, cie_hbm, cio_hbm, out_hbm, cie_v, cio_v, out_v, r2_v,
          sem_a, sem_b, sem_c, sem_d):
        cid = lax.axis_index("c")
        sid = lax.axis_index("s")
        wid = cid * 16 + sid
        t0 = wid * tpw
        cv = c_dim // 16

        pltpu.sync_copy(cie_hbm.at[pl.ds(t0, tpw)], cie_v)
        pltpu.sync_copy(cio_hbm.at[pl.ds(t0, tpw)], cio_v)

        def g_e(j, buf_slot, sem):
            return pltpu.async_copy(
                yb_hbm.at[cie_v.at[pl.ds(j * 16, 16)]], out_v.at[buf_slot],
                sem)

        def g_o(j, buf_slot, sem):
            return pltpu.async_copy(
                yb_hbm.at[cio_v.at[pl.ds(j * 16, 16)]], r2_v.at[buf_slot],
                sem)

        def add_rows(buf_slot):
            def tt_body(tt, carry):
                for cc in range(cv):
                    sl = pl.ds(cc * 16, 16)
                    out_v[buf_slot, tt, sl] = (out_v[buf_slot, tt, sl]
                                               + r2_v[buf_slot, tt, sl])
                return carry
            lax.fori_loop(0, 16, tt_body, 0)

        def chunk_body(jo, carry):
            j0 = jo * 2
            j1 = jo * 2 + 1
            e0 = g_e(j0, 0, sem_a)
            o0 = g_o(j0, 0, sem_c)
            e1 = g_e(j1, 1, sem_b)
            o1 = g_o(j1, 1, sem_d)
            e0.wait()
            o0.wait()
            add_rows(0)
            pltpu.sync_copy(out_v.at[0],
                            out_hbm.at[pl.ds(t0 + j0 * 16, 16)])
            e1.wait()
            o1.wait()
            add_rows(1)
            pltpu.sync_copy(out_v.at[1],
                            out_hbm.at[pl.ds(t0 + j1 * 16, 16)])
            return carry

        lax.fori_loop(0, nj // 2, chunk_body, 0)

    return k(yb, cie, cio)


def kernel(x, Wr, W1, b1, W2, b2):
    b, t, c = x.shape
    n = b * t
    cap = int(t / E * CAP_FACTOR)
    n_rows = E * b * cap
    rpe = b * cap
    yb_rows = n_rows + rpe
    buf_rows = yb_rows + 128

    xf = x.reshape(n, c)
    wr_pad = jnp.pad(Wr, ((0, 0), (0, LANES - E)))

    probs = _router(xf, wr_pad)
    buf, cie, cio, w_slot = _routing_dispatch(
        probs.reshape(-1), xf, n_tok=n, n_rows=n_rows, cap=cap, t_len=t,
        buf_rows=buf_rows)
    yb = _ffn(buf, W1, b1.reshape(E, 1, -1), W2, b2.reshape(E, 1, -1),
              w_slot.reshape(-1, 1), yb_rows=yb_rows)
    out = _combine(yb, cie, cio, n_tok=n)
    return out.reshape(b, t, c)
